# baseline (speedup 1.0000x reference)
import jax
import jax.numpy as jnp
from jax.experimental import pallas as pl
from jax.experimental.pallas import tpu as pltpu

_H = 200
_W = 200
_NF = 6
_NL = 16
_RB = 40
_NRB = _H // _RB
_LOGIT01 = -2.1972245773362196


def _occ_loss_kernel(traj_ref, gmask_ref, mask_ref, out_ref, cnt_ref, gau_ref, ms_ref):
    t = pl.program_id(0)
    rb = pl.program_id(1)

    @pl.when(jnp.logical_and(t == 0, rb == 0))
    def _init():
        ms_ref[0] = 0.0
        for i in range(_NF):
            cnt_ref[i] = 0.0
            gau_ref[i] = 0.0

    m = mask_ref[:, 0]
    mx = jnp.max(m, axis=0)
    occ = (mx > _LOGIT01).astype(jnp.float32)
    ms_ref[0] += jnp.sum(occ)

    r0 = (rb * _RB).astype(jnp.float32)
    rr = jax.lax.broadcasted_iota(jnp.int32, (_RB, _W), 0).astype(jnp.float32) + r0
    cc = jax.lax.broadcasted_iota(jnp.int32, (_RB, _W), 1).astype(jnp.float32)
    xg = jnp.trunc((cc - 100.0) * 0.5 + 0.25)
    yg = jnp.trunc((rr - 100.0) * 0.5 + 0.25)

    def add_future(i):
        px = traj_ref[i, 0]
        py = traj_ref[i, 1]
        dx = px - xg
        dy = py - yg
        d2 = dx * dx + dy * dy
        keep = (d2 < 25.0).astype(jnp.float32)
        w = occ * keep
        cnt_ref[i] += jnp.sum(w)
        gau_ref[i] += jnp.sum(jnp.exp(-0.5 * d2) * w)

    @pl.when(t > 0)
    def _mid():
        add_future(t - 1)

    @pl.when(t == _NF - 1)
    def _last():
        add_future(_NF - 1)

    @pl.when(jnp.logical_and(t == _NF - 1, rb == _NRB - 1))
    def _fin():
        num = 0.0
        den = 0.0
        for i in range(_NF):
            g = gmask_ref[i]
            valid_g = (cnt_ref[i] > 0.0).astype(jnp.float32) * g
            num += 0.5 * gau_ref[i] / 2.507 * valid_g
            den += valid_g
        loss = jnp.where(den > 0.0, num / jnp.maximum(den, 1.0), 0.0)
        loss = jnp.where(ms_ref[0] == 0.0, 0.0, loss)
        out_ref[0] = loss


def kernel(sdc_traj_all, sdc_planning_gt, sdc_planning_gt_mask, bev_mask, bev_target):
    traj = sdc_traj_all[0].astype(jnp.float32)
    gmask = (sdc_planning_gt_mask[0] != 0).astype(jnp.float32)
    bev = bev_mask[0]

    out = pl.pallas_call(
        _occ_loss_kernel,
        grid=(_NF, _NRB),
        in_specs=[
            pl.BlockSpec(memory_space=pltpu.SMEM),
            pl.BlockSpec(memory_space=pltpu.SMEM),
            pl.BlockSpec((_NL, 1, _RB, _W), lambda t, rb: (0, t, rb, 0)),
        ],
        out_specs=pl.BlockSpec(memory_space=pltpu.SMEM),
        out_shape=jax.ShapeDtypeStruct((1,), jnp.float32),
        scratch_shapes=[
            pltpu.SMEM((_NF,), jnp.float32),
            pltpu.SMEM((_NF,), jnp.float32),
            pltpu.SMEM((1,), jnp.float32),
        ],
    )(traj, gmask, bev)
    return out[0]


# speedup vs baseline: 2.1288x; 2.1288x over previous
import jax
import jax.numpy as jnp
from jax.experimental import pallas as pl
from jax.experimental.pallas import tpu as pltpu

_H = 200
_W = 200
_NF = 6
_NL = 16
_RB = 200
_NRB = _H // _RB
_LOGIT01 = -2.1972245773362196


def _occ_loss_kernel(traj_ref, gmask_ref, mask_ref, out_ref, cnt_ref, gau_ref, ms_ref):
    t = pl.program_id(0)
    rb = pl.program_id(1)

    @pl.when(jnp.logical_and(t == 0, rb == 0))
    def _init():
        ms_ref[0] = 0.0
        for i in range(_NF):
            cnt_ref[i] = 0.0
            gau_ref[i] = 0.0

    m = mask_ref[:, 0]
    mx = jnp.max(m, axis=0)
    occ = (mx > _LOGIT01).astype(jnp.float32)
    ms_ref[0] += jnp.sum(occ)

    r0 = (rb * _RB).astype(jnp.float32)
    rr = jax.lax.broadcasted_iota(jnp.int32, (_RB, _W), 0).astype(jnp.float32) + r0
    cc = jax.lax.broadcasted_iota(jnp.int32, (_RB, _W), 1).astype(jnp.float32)
    xg = jnp.trunc((cc - 100.0) * 0.5 + 0.25)
    yg = jnp.trunc((rr - 100.0) * 0.5 + 0.25)

    def add_future(i):
        px = traj_ref[i, 0]
        py = traj_ref[i, 1]
        dx = px - xg
        dy = py - yg
        d2 = dx * dx + dy * dy
        keep = (d2 < 25.0).astype(jnp.float32)
        w = occ * keep
        cnt_ref[i] += jnp.sum(w)
        gau_ref[i] += jnp.sum(jnp.exp(-0.5 * d2) * w)

    @pl.when(t > 0)
    def _mid():
        add_future(t - 1)

    @pl.when(t == _NF - 1)
    def _last():
        add_future(_NF - 1)

    @pl.when(jnp.logical_and(t == _NF - 1, rb == _NRB - 1))
    def _fin():
        num = 0.0
        den = 0.0
        for i in range(_NF):
            g = gmask_ref[i]
            valid_g = (cnt_ref[i] > 0.0).astype(jnp.float32) * g
            num += 0.5 * gau_ref[i] / 2.507 * valid_g
            den += valid_g
        loss = jnp.where(den > 0.0, num / jnp.maximum(den, 1.0), 0.0)
        loss = jnp.where(ms_ref[0] == 0.0, 0.0, loss)
        out_ref[0] = loss


def kernel(sdc_traj_all, sdc_planning_gt, sdc_planning_gt_mask, bev_mask, bev_target):
    traj = sdc_traj_all[0].astype(jnp.float32)
    gmask = (sdc_planning_gt_mask[0] != 0).astype(jnp.float32)
    bev = bev_mask[0]

    out = pl.pallas_call(
        _occ_loss_kernel,
        grid=(_NF, _NRB),
        in_specs=[
            pl.BlockSpec(memory_space=pltpu.SMEM),
            pl.BlockSpec(memory_space=pltpu.SMEM),
            pl.BlockSpec((_NL, 1, _RB, _W), lambda t, rb: (0, t, rb, 0)),
        ],
        out_specs=pl.BlockSpec(memory_space=pltpu.SMEM),
        out_shape=jax.ShapeDtypeStruct((1,), jnp.float32),
        scratch_shapes=[
            pltpu.SMEM((_NF,), jnp.float32),
            pltpu.SMEM((_NF,), jnp.float32),
            pltpu.SMEM((1,), jnp.float32),
        ],
    )(traj, gmask, bev)
    return out[0]
